# baseline (speedup 1.0000x reference)
import functools

import jax
import jax.numpy as jnp
from jax import lax
from jax.experimental import pallas as pl
from jax.experimental.pallas import tpu as pltpu
from jax.experimental.pallas import tpu_sc as plsc

_N = 10000
_E = 320000
_D = 128
_R = 8
_L = 3

_NC = 2
_NS = 16
_NW = _NC * _NS
_KA = 128
_KE = 128
_EPW = 10240
_NBA = _EPW // _KA
_NBW = _EPW // _KE
_NBW0 = 156
_NBW1 = 2 * _NBW - _NBW0
_EPAD = _EPW * _NW
_RN = _R * _N
_RNP = _RN + _KA
_CH = _RNP // _NS
_EPT = _EPAD // _NS
_NBT = _EPT // _KA
_NBN = 10
_BN = _N // _NBN
_FCH = 40
_NFCH = _N // _FCH
_FIT = -(-_NFCH // _NS)

_sc_mesh = plsc.VectorSubcoreMesh(core_axis_name="c", subcore_axis_name="s")


@functools.partial(
    pl.kernel,
    out_type=jax.ShapeDtypeStruct((_NW * _NBA, _KA), jnp.int32),
    mesh=_sc_mesh,
    scratch_types=[
        pltpu.VMEM_SHARED((_RNP,), jnp.int32),
        pltpu.VMEM((_NBT, _KA), jnp.int32),
        pltpu.VMEM((_NBA, _KA), jnp.int32),
        pltpu.VMEM((_NBA, _KA), jnp.int32),
        pltpu.VMEM((_CH,), jnp.int32),
        pltpu.VMEM((_KA,), jnp.int32),
        pltpu.SemaphoreType.DMA,
        pltpu.SemaphoreType.DMA,
        pltpu.SemaphoreType.DMA,
        pltpu.SemaphoreType.DMA,
    ],
)
def _count_kernel(rdstc_hbm, rdstg_hbm, cnt_hbm,
                  cnt_sh, rc_v, rg_v, val_v, zv, ones_v, semp0, semp1, sema, semg):
    sid = lax.axis_index("s")
    cid = lax.axis_index("c")
    wid = sid * _NC + cid
    z16 = jnp.zeros((16,), jnp.int32)
    o16 = jnp.ones((16,), jnp.int32)
    for j in range(_KA // 16):
        ones_v[pl.ds(j * 16, 16)] = o16
    for j in range(_CH // 16):
        zv[pl.ds(j * 16, 16)] = z16
    pltpu.async_copy(rdstc_hbm.at[pl.ds(sid * _NBT, _NBT)], rc_v, semp0)
    pltpu.async_copy(rdstg_hbm.at[pl.ds(wid * _NBA, _NBA)], rg_v, semp1)
    pltpu.sync_copy(zv, cnt_sh.at[pl.ds(sid * _CH, _CH)])
    pltpu.make_async_copy(rdstc_hbm.at[pl.ds(sid * _NBT, _NBT)], rc_v, semp0).wait()
    pltpu.make_async_copy(rdstg_hbm.at[pl.ds(wid * _NBA, _NBA)], rg_v, semp1).wait()
    plsc.subcore_barrier()

    def count_fire(k, carry):
        pltpu.async_copy(ones_v, cnt_sh.at[rc_v.at[k]], sema, add=True)
        return carry

    lax.fori_loop(0, _NBT, count_fire, 0)

    def count_drain(k, carry):
        pltpu.make_async_copy(ones_v, cnt_sh.at[rc_v.at[k]], sema).wait()
        return carry

    lax.fori_loop(0, _NBT, count_drain, 0)
    plsc.subcore_barrier()

    def gather_fire(k, carry):
        pltpu.async_copy(cnt_sh.at[rg_v.at[k]], val_v.at[k], semg)
        return carry

    lax.fori_loop(0, _NBA, gather_fire, 0)

    def gather_drain(k, carry):
        pltpu.make_async_copy(cnt_sh.at[rg_v.at[k]], val_v.at[k], semg).wait()
        return carry

    lax.fori_loop(0, _NBA, gather_drain, 0)
    pltpu.sync_copy(val_v, cnt_hbm.at[pl.ds(wid * _NBA, _NBA), :])


@functools.partial(
    pl.kernel,
    out_type=jax.ShapeDtypeStruct((_NC, _N, _D), jnp.float32),
    mesh=_sc_mesh,
    scratch_types=[
        pltpu.VMEM_SHARED((_N, _D), jnp.float32),
        pltpu.VMEM((_KE, _D), jnp.float32),
        pltpu.VMEM((_KE, _D), jnp.float32),
        [pltpu.VMEM((3, _KE), jnp.int32)] * 4,
        pltpu.VMEM((_FCH, _D), jnp.float32),
        [pltpu.SemaphoreType.DMA] * 2,
        [pltpu.SemaphoreType.DMA] * 2,
        [pltpu.SemaphoreType.DMA] * 4,
    ],
)
def _agg_kernel(hflat_hbm, packed_hbm, agg_hbm,
                acc_sh, rows0, rows1, ebs, zr, gsem, ssem, isem):
    sid = lax.axis_index("s")
    cid = lax.axis_index("c")
    rows = [rows0, rows1]
    blk0 = jnp.where(cid == 0, sid * _NBW0, _NS * _NBW0 + sid * _NBW1)
    z16 = jnp.zeros((16,), jnp.float32)
    for rr in range(_FCH):
        for c in range(_D // 16):
            zr[rr, pl.ds(c * 16, 16)] = z16

    def start_idx(k, i):
        pltpu.async_copy(packed_hbm.at[blk0 + k], ebs[i], isem[i])

    def wait_idx(k, i):
        pltpu.make_async_copy(packed_hbm.at[blk0 + k], ebs[i], isem[i]).wait()

    def start_gather(i, u):
        pltpu.async_copy(hflat_hbm.at[ebs[i].at[0]], rows[u], gsem[u])

    def wait_gather(i, u):
        pltpu.make_async_copy(hflat_hbm.at[ebs[i].at[0]], rows[u], gsem[u]).wait()

    def start_scatter(i, u):
        pltpu.async_copy(rows[u], acc_sh.at[ebs[i].at[1]], ssem[u], add=True)

    def wait_scatter(i, u):
        pltpu.make_async_copy(rows[u], acc_sh.at[ebs[i].at[1]], ssem[u]).wait()

    def scale_rows(i, u):
        def scale_step(jj, c2):
            ci = ebs[i][2, pl.ds(jj * 16, 16)]
            cf = ci.astype(jnp.float32)
            sv = jnp.where(ci >= 1, 1.0 / jnp.maximum(cf, 1.0), 0.0)
            for uu in range(16):
                j = jj * 16 + uu
                s = sv[uu]
                for c in range(_D // 16):
                    rows[u][j, pl.ds(c * 16, 16)] = rows[u][j, pl.ds(c * 16, 16)] * s
            return c2

        lax.fori_loop(0, _KE // 16, scale_step, 0)

    def pipeline(nbw):
        for k in range(2):
            start_idx(k, k)
        wait_idx(0, 0)
        start_gather(0, 0)
        wait_idx(1, 1)
        start_gather(1, 1)

        def quad(g, carry):
            b = 4 * g
            for su in range(4):
                k = b + su
                u = su % 2
                i = su % 4
                @pl.when(k >= 1)
                def _():
                    wait_scatter((su - 1) % 4, (su - 1) % 2)
                @pl.when((k + 1 >= 2) & (k + 1 < nbw))
                def _():
                    wait_idx(k + 1, (su + 1) % 4)
                    start_gather((su + 1) % 4, (su + 1) % 2)
                wait_gather(i, u)
                scale_rows(i, u)
                start_scatter(i, u)
                @pl.when(k + 2 < nbw)
                def _():
                    start_idx(k + 2, (su + 2) % 4)
            return carry

        lax.fori_loop(0, nbw // 4, quad, 0)
        wait_scatter((nbw - 1) % 4, (nbw - 1) % 2)

    for k in range(_FIT):
        chunk = k * _NS + sid
        @pl.when(chunk < _NFCH)
        def _():
            r0 = pl.multiple_of(chunk * _FCH, 8)
            pltpu.sync_copy(zr, acc_sh.at[pl.ds(r0, _FCH), :])
    plsc.subcore_barrier()

    @pl.when(cid == 0)
    def _():
        pipeline(_NBW0)

    @pl.when(cid == 1)
    def _():
        pipeline(_NBW1)

    plsc.subcore_barrier()

    for k in range(_FIT):
        chunk = k * _NS + sid
        @pl.when(chunk < _NFCH)
        def _():
            r0 = pl.multiple_of(chunk * _FCH, 8)
            pltpu.sync_copy(acc_sh.at[pl.ds(r0, _FCH), :], zr)
            pltpu.sync_copy(zr, agg_hbm.at[cid, pl.ds(r0, _FCH), :])


def _mm_body(h_ref, w_ref, out_ref):
    out_ref[0] = jnp.dot(h_ref[...], w_ref[0], preferred_element_type=jnp.float32)


def _matmul(h, w9):
    return pl.pallas_call(
        _mm_body,
        grid=(_R + 1, _NBN),
        in_specs=[
            pl.BlockSpec((_BN, _D), lambda r, i: (i, 0)),
            pl.BlockSpec((1, _D, _D), lambda r, i: (r, 0, 0)),
        ],
        out_specs=pl.BlockSpec((1, _BN, _D), lambda r, i: (r, i, 0)),
        out_shape=jax.ShapeDtypeStruct((_R + 1, _N, _D), jnp.float32),
    )(h, w9)


def _comb_body(h9_ref, agg_ref, b_ref, out_ref, *, relu):
    v = h9_ref[0] + agg_ref[0] + agg_ref[1] + b_ref[...]
    if relu:
        v = jnp.maximum(v, 0.0)
    out_ref[...] = v


def _combine(h9, agg, bias2d, relu):
    return pl.pallas_call(
        functools.partial(_comb_body, relu=relu),
        grid=(_NBN,),
        in_specs=[
            pl.BlockSpec((1, _BN, _D), lambda i: (_R, i, 0)),
            pl.BlockSpec((_NC, _BN, _D), lambda i: (0, i, 0)),
            pl.BlockSpec((1, _D), lambda i: (0, 0)),
        ],
        out_specs=pl.BlockSpec((_BN, _D), lambda i: (i, 0)),
        out_shape=jax.ShapeDtypeStruct((_N, _D), jnp.float32),
    )(h9, agg, bias2d)


def kernel(x, edge_index, edge_type, weights, roots, biases):
    src = edge_index[0].astype(jnp.int32)
    dst = edge_index[1].astype(jnp.int32)
    et = edge_type.astype(jnp.int32)
    gidx = et * _N + src
    rdst = et * _N + dst
    pad = _EPAD - _E
    gidx_p = jnp.pad(gidx, (0, pad))
    dst_p = jnp.pad(dst, (0, pad))
    rdstc = jnp.pad(rdst, (0, pad), constant_values=_RN)
    rdstg = jnp.pad(rdst, (0, pad), constant_values=_RN + 1)
    w9 = jnp.concatenate([weights, roots[:, None]], axis=1)

    cnt = _count_kernel(rdstc.reshape(_NS * _NBT, _KA),
                        rdstg.reshape(_NW * _NBA, _KA))
    nblk = _NW * _NBW
    packed = jnp.stack(
        [gidx_p.reshape(nblk, _KE), dst_p.reshape(nblk, _KE),
         cnt.reshape(nblk, _KE)],
        axis=1)
    h = x
    for l in range(_L):
        h9 = _matmul(h, w9[l])
        agg = _agg_kernel(h9.reshape(((_R + 1) * _N, _D)), packed)
        h = _combine(h9, agg, biases[l][None], relu=(l < _L - 1))
    return h


# speedup vs baseline: 1.0564x; 1.0564x over previous
import functools

import jax
import jax.numpy as jnp
from jax import lax
from jax.experimental import pallas as pl
from jax.experimental.pallas import tpu as pltpu
from jax.experimental.pallas import tpu_sc as plsc

_N = 10000
_E = 320000
_D = 128
_R = 8
_L = 3

_NC = 2
_NS = 16
_NW = _NC * _NS
_KA = 128
_KE = 128
_EPW = 10240
_NBA = _EPW // _KA
_NBW = _EPW // _KE
_NBW0 = 152
_NBW1 = 2 * _NBW - _NBW0
_EPAD = _EPW * _NW
_RN = _R * _N
_RNP = _RN + _KA
_CH = _RNP // _NS
_EPT = _EPAD // _NS
_NBT = _EPT // _KA
_NBN = 10
_BN = _N // _NBN
_FCH = 40
_NFCH = _N // _FCH
_FIT = -(-_NFCH // _NS)

_sc_mesh = plsc.VectorSubcoreMesh(core_axis_name="c", subcore_axis_name="s")


@functools.partial(
    pl.kernel,
    out_type=jax.ShapeDtypeStruct((_NW * _NBA, _KA), jnp.int32),
    mesh=_sc_mesh,
    scratch_types=[
        pltpu.VMEM_SHARED((_RNP,), jnp.int32),
        pltpu.VMEM((_NBT, _KA), jnp.int32),
        pltpu.VMEM((_NBA, _KA), jnp.int32),
        pltpu.VMEM((_NBA, _KA), jnp.int32),
        pltpu.VMEM((_CH,), jnp.int32),
        pltpu.VMEM((_KA,), jnp.int32),
        pltpu.SemaphoreType.DMA,
        pltpu.SemaphoreType.DMA,
        pltpu.SemaphoreType.DMA,
        pltpu.SemaphoreType.DMA,
    ],
)
def _count_kernel(rdstc_hbm, rdstg_hbm, cnt_hbm,
                  cnt_sh, rc_v, rg_v, val_v, zv, ones_v, semp0, semp1, sema, semg):
    sid = lax.axis_index("s")
    cid = lax.axis_index("c")
    wid = sid * _NC + cid
    z16 = jnp.zeros((16,), jnp.int32)
    o16 = jnp.ones((16,), jnp.int32)
    for j in range(_KA // 16):
        ones_v[pl.ds(j * 16, 16)] = o16
    for j in range(_CH // 16):
        zv[pl.ds(j * 16, 16)] = z16
    pltpu.async_copy(rdstc_hbm.at[pl.ds(sid * _NBT, _NBT)], rc_v, semp0)
    pltpu.async_copy(rdstg_hbm.at[pl.ds(wid * _NBA, _NBA)], rg_v, semp1)
    pltpu.sync_copy(zv, cnt_sh.at[pl.ds(sid * _CH, _CH)])
    pltpu.make_async_copy(rdstc_hbm.at[pl.ds(sid * _NBT, _NBT)], rc_v, semp0).wait()
    pltpu.make_async_copy(rdstg_hbm.at[pl.ds(wid * _NBA, _NBA)], rg_v, semp1).wait()
    plsc.subcore_barrier()

    def count_fire(k, carry):
        pltpu.async_copy(ones_v, cnt_sh.at[rc_v.at[k]], sema, add=True)
        return carry

    lax.fori_loop(0, _NBT, count_fire, 0)

    def count_drain(k, carry):
        pltpu.make_async_copy(ones_v, cnt_sh.at[rc_v.at[k]], sema).wait()
        return carry

    lax.fori_loop(0, _NBT, count_drain, 0)
    plsc.subcore_barrier()

    def gather_fire(k, carry):
        pltpu.async_copy(cnt_sh.at[rg_v.at[k]], val_v.at[k], semg)
        return carry

    lax.fori_loop(0, _NBA, gather_fire, 0)

    def gather_drain(k, carry):
        pltpu.make_async_copy(cnt_sh.at[rg_v.at[k]], val_v.at[k], semg).wait()
        return carry

    lax.fori_loop(0, _NBA, gather_drain, 0)
    pltpu.sync_copy(val_v, cnt_hbm.at[pl.ds(wid * _NBA, _NBA), :])


@functools.partial(
    pl.kernel,
    out_type=jax.ShapeDtypeStruct((_NC, _N, _D), jnp.float32),
    mesh=_sc_mesh,
    scratch_types=[
        pltpu.VMEM_SHARED((_N, _D), jnp.float32),
        pltpu.VMEM((_KE, _D), jnp.float32),
        pltpu.VMEM((_KE, _D), jnp.float32),
        [pltpu.VMEM((3, _KE), jnp.int32)] * 4,
        pltpu.VMEM((_FCH, _D), jnp.float32),
        [pltpu.SemaphoreType.DMA] * 2,
        [pltpu.SemaphoreType.DMA] * 2,
        [pltpu.SemaphoreType.DMA] * 4,
    ],
)
def _agg_kernel(hflat_hbm, packed_hbm, agg_hbm,
                acc_sh, rows0, rows1, ebs, zr, gsem, ssem, isem):
    sid = lax.axis_index("s")
    cid = lax.axis_index("c")
    rows = [rows0, rows1]
    blk0 = jnp.where(cid == 0, sid * _NBW0, _NS * _NBW0 + sid * _NBW1)
    z16 = jnp.zeros((16,), jnp.float32)
    for rr in range(_FCH):
        for c in range(_D // 16):
            zr[rr, pl.ds(c * 16, 16)] = z16

    def start_idx(k, i):
        pltpu.async_copy(packed_hbm.at[blk0 + k], ebs[i], isem[i])

    def wait_idx(k, i):
        pltpu.make_async_copy(packed_hbm.at[blk0 + k], ebs[i], isem[i]).wait()

    def start_gather(i, u):
        pltpu.async_copy(hflat_hbm.at[ebs[i].at[0]], rows[u], gsem[u])

    def wait_gather(i, u):
        pltpu.make_async_copy(hflat_hbm.at[ebs[i].at[0]], rows[u], gsem[u]).wait()

    def start_scatter(i, u):
        pltpu.async_copy(rows[u], acc_sh.at[ebs[i].at[1]], ssem[u], add=True)

    def wait_scatter(i, u):
        pltpu.make_async_copy(rows[u], acc_sh.at[ebs[i].at[1]], ssem[u]).wait()

    def scale_rows(i, u):
        def scale_step(jj, c2):
            ci = ebs[i][2, pl.ds(jj * 16, 16)]
            cf = ci.astype(jnp.float32)
            sv = jnp.where(ci >= 1, 1.0 / jnp.maximum(cf, 1.0), 0.0)
            for uu in range(16):
                j = jj * 16 + uu
                s = sv[uu]
                for c in range(_D // 16):
                    rows[u][j, pl.ds(c * 16, 16)] = rows[u][j, pl.ds(c * 16, 16)] * s
            return c2

        lax.fori_loop(0, _KE // 16, scale_step, 0)

    def pipeline(nbw):
        for k in range(2):
            start_idx(k, k)
        wait_idx(0, 0)
        start_gather(0, 0)
        wait_idx(1, 1)
        start_gather(1, 1)

        def quad(g, carry):
            b = 4 * g
            for su in range(4):
                k = b + su
                u = su % 2
                i = su % 4
                @pl.when(k >= 1)
                def _():
                    wait_scatter((su - 1) % 4, (su - 1) % 2)
                @pl.when((k + 1 >= 2) & (k + 1 < nbw))
                def _():
                    wait_idx(k + 1, (su + 1) % 4)
                    start_gather((su + 1) % 4, (su + 1) % 2)
                wait_gather(i, u)
                scale_rows(i, u)
                start_scatter(i, u)
                @pl.when(k + 2 < nbw)
                def _():
                    start_idx(k + 2, (su + 2) % 4)
            return carry

        lax.fori_loop(0, nbw // 4, quad, 0)
        wait_scatter((nbw - 1) % 4, (nbw - 1) % 2)

    for k in range(_FIT):
        chunk = k * _NS + sid
        @pl.when(chunk < _NFCH)
        def _():
            r0 = pl.multiple_of(chunk * _FCH, 8)
            pltpu.sync_copy(zr, acc_sh.at[pl.ds(r0, _FCH), :])
    plsc.subcore_barrier()

    @pl.when(cid == 0)
    def _():
        pipeline(_NBW0)

    @pl.when(cid == 1)
    def _():
        pipeline(_NBW1)

    plsc.subcore_barrier()

    for k in range(_FIT):
        chunk = k * _NS + sid
        @pl.when(chunk < _NFCH)
        def _():
            r0 = pl.multiple_of(chunk * _FCH, 8)
            pltpu.sync_copy(acc_sh.at[pl.ds(r0, _FCH), :], zr)
            pltpu.sync_copy(zr, agg_hbm.at[cid, pl.ds(r0, _FCH), :])


def _mm_body(h_ref, w_ref, out_ref):
    out_ref[0] = jnp.dot(h_ref[...], w_ref[0], preferred_element_type=jnp.float32)


def _matmul(h, w9):
    return pl.pallas_call(
        _mm_body,
        grid=(_R + 1, _NBN),
        in_specs=[
            pl.BlockSpec((_BN, _D), lambda r, i: (i, 0)),
            pl.BlockSpec((1, _D, _D), lambda r, i: (r, 0, 0)),
        ],
        out_specs=pl.BlockSpec((1, _BN, _D), lambda r, i: (r, i, 0)),
        out_shape=jax.ShapeDtypeStruct((_R + 1, _N, _D), jnp.float32),
    )(h, w9)


def _comb_body(h9_ref, agg_ref, b_ref, out_ref, *, relu):
    v = h9_ref[0] + agg_ref[0] + agg_ref[1] + b_ref[...]
    if relu:
        v = jnp.maximum(v, 0.0)
    out_ref[...] = v


def _combine(h9, agg, bias2d, relu):
    return pl.pallas_call(
        functools.partial(_comb_body, relu=relu),
        grid=(_NBN,),
        in_specs=[
            pl.BlockSpec((1, _BN, _D), lambda i: (_R, i, 0)),
            pl.BlockSpec((_NC, _BN, _D), lambda i: (0, i, 0)),
            pl.BlockSpec((1, _D), lambda i: (0, 0)),
        ],
        out_specs=pl.BlockSpec((_BN, _D), lambda i: (i, 0)),
        out_shape=jax.ShapeDtypeStruct((_N, _D), jnp.float32),
    )(h9, agg, bias2d)


def kernel(x, edge_index, edge_type, weights, roots, biases):
    src = edge_index[0].astype(jnp.int32)
    dst = edge_index[1].astype(jnp.int32)
    et = edge_type.astype(jnp.int32)
    gidx = et * _N + src
    rdst = et * _N + dst
    pad = _EPAD - _E
    gidx_p = jnp.pad(gidx, (0, pad))
    dst_p = jnp.pad(dst, (0, pad))
    rdstc = jnp.pad(rdst, (0, pad), constant_values=_RN)
    rdstg = jnp.pad(rdst, (0, pad), constant_values=_RN + 1)
    w9 = jnp.concatenate([weights, roots[:, None]], axis=1)

    cnt = _count_kernel(rdstc.reshape(_NS * _NBT, _KA),
                        rdstg.reshape(_NW * _NBA, _KA))
    nblk = _NW * _NBW
    packed = jnp.stack(
        [gidx_p.reshape(nblk, _KE), dst_p.reshape(nblk, _KE),
         cnt.reshape(nblk, _KE)],
        axis=1)
    h = x
    for l in range(_L):
        h9 = _matmul(h, w9[l])
        agg = _agg_kernel(h9.reshape(((_R + 1) * _N, _D)), packed)
        h = _combine(h9, agg, biases[l][None], relu=(l < _L - 1))
    return h
